# Initial kernel scaffold; baseline (speedup 1.0000x reference)
import functools

import jax
import jax.numpy as jnp
from jax import lax
from jax.experimental import pallas as pl
from jax.experimental.pallas import tpu as pltpu
from jax.experimental.pallas import tpu_sc as plsc

NC, NS, LANES = 2, 16, 16
NW = NC * NS

IB = 80


def _mesh():
    return plsc.VectorSubcoreMesh(
        core_axis_name="c", subcore_axis_name="s",
        num_cores=NC, num_subcores=NS)


def _gather_motif(motif, src2, dst2):
    N, M = motif.shape
    E = src2.shape[0] * IB
    EW = E // NW
    CH = 2000
    KB = CH // IB

    @functools.partial(
        pl.kernel,
        out_type=(jax.ShapeDtypeStruct((E, M), jnp.float32),
                  jax.ShapeDtypeStruct((E, M), jnp.float32)),
        mesh=_mesh(),
        scratch_types=[
            pltpu.VMEM((KB, IB), jnp.int32),
            pltpu.VMEM((KB, IB), jnp.int32),
            pltpu.VMEM((CH, M), jnp.float32),
            pltpu.VMEM((CH, M), jnp.float32),
            pltpu.SemaphoreType.DMA,
            pltpu.SemaphoreType.DMA,
        ],
    )
    def k(motif_hbm, src_hbm, dst_hbm, mu_hbm, mv_hbm,
          sidx, didx, mrow, nrow, sem1, sem2):
        wid = lax.axis_index("s") * NC + lax.axis_index("c")
        base = wid * EW

        @pl.loop(0, EW // CH)
        def chunk(kk):
            off = pl.multiple_of(base + kk * CH, IB)
            orow = off // IB
            pltpu.sync_copy(src_hbm.at[pl.ds(orow, KB), :], sidx)
            pltpu.sync_copy(dst_hbm.at[pl.ds(orow, KB), :], didx)
            for j in range(KB):
                pltpu.async_copy(motif_hbm.at[sidx.at[j]],
                                 mrow.at[pl.ds(j * IB, IB), :], sem1)
                pltpu.async_copy(motif_hbm.at[didx.at[j]],
                                 nrow.at[pl.ds(j * IB, IB), :], sem2)
            for j in range(KB):
                pltpu.make_async_copy(motif_hbm.at[sidx.at[j]],
                                      mrow.at[pl.ds(j * IB, IB), :], sem1).wait()
                pltpu.make_async_copy(motif_hbm.at[didx.at[j]],
                                      nrow.at[pl.ds(j * IB, IB), :], sem2).wait()
            pltpu.sync_copy(mrow, mu_hbm.at[pl.ds(off, CH), :])
            pltpu.sync_copy(nrow, mv_hbm.at[pl.ds(off, CH), :])

    return k(motif, src2, dst2)


def _gate_tc(mu, mv, G1, g1b, G2, g2b):
    E, M = mu.shape
    BE = 3200
    grid = E // BE

    def body(mu_ref, mv_ref, G1_ref, g1b_ref, g2r_ref, g2b_ref, out_ref):
        a = mu_ref[...]
        b = mv_ref[...]
        feats = jnp.concatenate([a, b, jnp.abs(a - b), a * b], axis=1)
        h = jnp.dot(feats, G1_ref[...], preferred_element_type=jnp.float32)
        h = jnp.maximum(h + g1b_ref[...], 0.0)
        g = jnp.sum(h * g2r_ref[...], axis=1, keepdims=True) + g2b_ref[0, 0]
        g = jax.nn.sigmoid(g)
        out_ref[...] = jnp.clip(g, 0.0, 1.0)

    return pl.pallas_call(
        body,
        grid=(grid,),
        in_specs=[
            pl.BlockSpec((BE, M), lambda i: (i, 0)),
            pl.BlockSpec((BE, M), lambda i: (i, 0)),
            pl.BlockSpec((4 * M, 64), lambda i: (0, 0)),
            pl.BlockSpec((1, 64), lambda i: (0, 0)),
            pl.BlockSpec((1, 64), lambda i: (0, 0)),
            pl.BlockSpec((1, 1), lambda i: (0, 0)),
        ],
        out_specs=pl.BlockSpec((BE, 1), lambda i: (i, 0)),
        out_shape=jax.ShapeDtypeStruct((E, 1), jnp.float32),
    )(mu, mv, G1, g1b.reshape(1, 64), G2.reshape(1, 64), g2b.reshape(1, 1))


def _deg_sc(col2, gate2d, zeros1, NPAD):
    E = col2.shape[0] * IB
    EW = E // NW
    CH = 2000
    KB = CH // IB
    SL = NPAD // NS

    @functools.partial(
        pl.kernel,
        out_type=jax.ShapeDtypeStruct((NC, NPAD, 1), jnp.float32),
        mesh=_mesh(),
        scratch_types=[
            pltpu.VMEM((KB, IB), jnp.int32),
            pltpu.VMEM((CH, 1), jnp.float32),
            pltpu.VMEM_SHARED((NPAD, 1), jnp.float32),
        ],
    )
    def k(col_hbm, gate_hbm, z_hbm, out_hbm, cidx, gv, acc):
        cid = lax.axis_index("c")
        sid = lax.axis_index("s")
        wid = sid * NC + cid
        base = wid * EW
        r0 = sid * SL
        pltpu.sync_copy(z_hbm.at[pl.ds(r0, SL), :], acc.at[pl.ds(r0, SL), :])
        plsc.subcore_barrier()

        @pl.loop(0, EW // CH)
        def chunk(kk):
            off = pl.multiple_of(base + kk * CH, IB)
            orow = off // IB
            pltpu.sync_copy(col_hbm.at[pl.ds(orow, KB), :], cidx)
            pltpu.sync_copy(gate_hbm.at[pl.ds(off, CH), :], gv)
            for j in range(KB):
                pltpu.sync_copy(gv.at[pl.ds(j * IB, IB), :],
                                acc.at[cidx.at[j]], add=True)

        plsc.subcore_barrier()
        pltpu.sync_copy(acc.at[pl.ds(r0, SL), :],
                        out_hbm.at[cid, pl.ds(r0, SL), :])

    return k(col2, gate2d, zeros1)


def _dense1_tc(x, W1, res1_W, res1_b, degp, NPAD):
    N = x.shape[0]
    H = W1.shape[1]

    def body(x_ref, W1_ref, rW_ref, rb_ref, degp_ref, ys1_ref, xres_ref, dinv_ref):
        deg = degp_ref[0] + degp_ref[1] + 1.0
        dinv = lax.rsqrt(deg)
        dinv_ref[...] = dinv
        xw = jnp.dot(x_ref[...], W1_ref[...], preferred_element_type=jnp.float32)
        ys1_ref[...] = xw * dinv[:N]
        xres_ref[...] = (jnp.dot(x_ref[...], rW_ref[...],
                                 preferred_element_type=jnp.float32) + rb_ref[...])

    return pl.pallas_call(
        body,
        out_shape=(jax.ShapeDtypeStruct((N, H), jnp.float32),
                   jax.ShapeDtypeStruct((N, H), jnp.float32),
                   jax.ShapeDtypeStruct((NPAD, 1), jnp.float32)),
    )(x, W1, res1_W, res1_b.reshape(1, H), degp)


def _conv_sc(ys, row2, col2, gate_flat, dinv_flat, zeros2, NPAD):
    N, H = ys.shape
    E = gate_flat.shape[0]
    EW = E // NW
    CH = 400
    KB = CH // IB
    SL = NPAD // NS

    @functools.partial(
        pl.kernel,
        out_type=jax.ShapeDtypeStruct((NC, NPAD, H), jnp.float32),
        mesh=_mesh(),
        scratch_types=[
            pltpu.VMEM((NPAD,), jnp.float32),
            pltpu.VMEM((KB, IB), jnp.int32),
            pltpu.VMEM((KB, IB), jnp.int32),
            pltpu.VMEM((CH,), jnp.float32),
            pltpu.VMEM((CH,), jnp.float32),
            pltpu.VMEM((CH, H), jnp.float32),
            pltpu.VMEM_SHARED((NPAD, H), jnp.float32),
            pltpu.SemaphoreType.DMA,
        ],
    )
    def k(ys_hbm, row_hbm, col_hbm, gate_hbm, dinv_hbm, z_hbm, out_hbm,
          dinv_v, rid, cidx, gv, wv, rows, acc, sem):
        cid_ax = lax.axis_index("c")
        sid = lax.axis_index("s")
        wid = sid * NC + cid_ax
        base = wid * EW
        r0 = sid * SL
        pltpu.sync_copy(dinv_hbm, dinv_v)
        pltpu.sync_copy(z_hbm.at[pl.ds(r0, SL), :], acc.at[pl.ds(r0, SL), :])
        plsc.subcore_barrier()

        @pl.loop(0, EW // CH)
        def chunk(kk):
            off = pl.multiple_of(base + kk * CH, IB)
            orow = off // IB
            pltpu.sync_copy(row_hbm.at[pl.ds(orow, KB), :], rid)
            pltpu.sync_copy(col_hbm.at[pl.ds(orow, KB), :], cidx)
            pltpu.sync_copy(gate_hbm.at[pl.ds(off, CH)], gv)
            for j in range(KB):
                pltpu.async_copy(ys_hbm.at[rid.at[j]],
                                 rows.at[pl.ds(j * IB, IB), :], sem)
            for j in range(KB):
                pltpu.make_async_copy(ys_hbm.at[rid.at[j]],
                                      rows.at[pl.ds(j * IB, IB), :], sem).wait()

            @pl.loop(0, CH // LANES)
            def wloop(t):
                o = pl.multiple_of(t * LANES, 8)
                jj = o // IB
                ci = cidx[jj, pl.ds(o - jj * IB, LANES)]
                dv = plsc.load_gather(dinv_v, [ci])
                wv[pl.ds(o, LANES)] = gv[pl.ds(o, LANES)] * dv

            @plsc.parallel_loop(0, CH, 1, unroll=4)
            def scale(e):
                s = wv[e]
                for j in range(H // LANES):
                    rows[e, pl.ds(j * LANES, LANES)] = (
                        rows[e, pl.ds(j * LANES, LANES)] * s)

            for j in range(KB):
                pltpu.sync_copy(rows.at[pl.ds(j * IB, IB), :],
                                acc.at[cidx.at[j]], add=True)

        plsc.subcore_barrier()
        pltpu.sync_copy(acc.at[pl.ds(r0, SL), :],
                        out_hbm.at[cid_ax, pl.ds(r0, SL), :])

    return k(ys, row2, col2, gate_flat, dinv_flat, zeros2)


def _post1_tc(accp, ys1, dinvc, b1, ln1_g, ln1_b, xres, W2):
    N, H = ys1.shape

    def body(accp_ref, ys1_ref, dinv_ref, b1_ref, g_ref, bb_ref, xres_ref,
             W2_ref, x1_ref, ys2_ref):
        dinv = dinv_ref[pl.ds(0, N), :]
        h = (accp_ref[0, pl.ds(0, N), :] + accp_ref[1, pl.ds(0, N), :]
             + ys1_ref[...] * dinv + b1_ref[...])
        mu = jnp.mean(h, axis=1, keepdims=True)
        var = jnp.mean((h - mu) ** 2, axis=1, keepdims=True)
        h = (h - mu) / jnp.sqrt(var + 1e-5) * g_ref[...] + bb_ref[...]
        h = jnp.maximum(h, 0.0)
        x1 = xres_ref[...] + h
        x1_ref[...] = x1
        ys2_ref[...] = jnp.dot(x1, W2_ref[...],
                               preferred_element_type=jnp.float32) * dinv

    return pl.pallas_call(
        body,
        out_shape=(jax.ShapeDtypeStruct((N, H), jnp.float32),
                   jax.ShapeDtypeStruct((N, H), jnp.float32)),
    )(accp, ys1, dinvc, b1.reshape(1, H), ln1_g.reshape(1, H),
      ln1_b.reshape(1, H), xres, W2)


def _final_tc(accp, ys2, dinvc, b2, ln2_g, ln2_b, x1, head_W, head_b):
    N, H = ys2.shape
    OUT = head_W.shape[1]

    def body(accp_ref, ys2_ref, dinv_ref, b2_ref, g_ref, bb_ref, x1_ref,
             hW_ref, hb_ref, out_ref):
        dinv = dinv_ref[pl.ds(0, N), :]
        h = (accp_ref[0, pl.ds(0, N), :] + accp_ref[1, pl.ds(0, N), :]
             + ys2_ref[...] * dinv + b2_ref[...])
        mu = jnp.mean(h, axis=1, keepdims=True)
        var = jnp.mean((h - mu) ** 2, axis=1, keepdims=True)
        h = (h - mu) / jnp.sqrt(var + 1e-5) * g_ref[...] + bb_ref[...]
        h = jnp.maximum(h, 0.0)
        x2 = x1_ref[...] + h
        out_ref[...] = (jnp.dot(x2, hW_ref[...],
                                preferred_element_type=jnp.float32) + hb_ref[...])

    return pl.pallas_call(
        body,
        out_shape=jax.ShapeDtypeStruct((N, OUT), jnp.float32),
    )(accp, ys2, dinvc, b2.reshape(1, H), ln2_g.reshape(1, H),
      ln2_b.reshape(1, H), x1, head_W, head_b.reshape(1, OUT))


def kernel(x, edge_index, motif_x, G1, g1b, G2, g2b, W1, b1, ln1_g, ln1_b,
           res1_W, res1_b, W2, b2, ln2_g, ln2_b, head_W, head_b):
    N = x.shape[0]
    E = edge_index.shape[1]
    H = W1.shape[1]
    NPAD = ((N + NS * 8 - 1) // (NS * 8)) * (NS * 8)

    row = edge_index[0]
    col = edge_index[1]
    row2 = row.reshape(E // IB, IB)
    col2 = col.reshape(E // IB, IB)
    zeros1 = jnp.zeros((NPAD, 1), jnp.float32)
    zeros2 = jnp.zeros((NPAD, H), jnp.float32)

    mu, mv = _gather_motif(motif_x, row2, col2)
    gate2d = _gate_tc(mu, mv, G1, g1b, G2, g2b)
    gate_flat = gate2d.reshape(E)
    degp = _deg_sc(col2, gate2d, zeros1, NPAD)
    ys1, xres, dinvc = _dense1_tc(x, W1, res1_W, res1_b, degp, NPAD)
    dinv_flat = dinvc.reshape(NPAD)
    accp1 = _conv_sc(ys1, row2, col2, gate_flat, dinv_flat, zeros2, NPAD)
    x1, ys2 = _post1_tc(accp1, ys1, dinvc, b1, ln1_g, ln1_b, xres, W2)
    accp2 = _conv_sc(ys2, row2, col2, gate_flat, dinv_flat, zeros2, NPAD)
    return _final_tc(accp2, ys2, dinvc, b2, ln2_g, ln2_b, x1, head_W, head_b)


# speedup vs baseline: 9.7730x; 9.7730x over previous
import functools

import jax
import jax.numpy as jnp
from jax import lax
from jax.experimental import pallas as pl
from jax.experimental.pallas import tpu as pltpu
from jax.experimental.pallas import tpu_sc as plsc

NC, NS, LANES = 2, 16, 16
NW = NC * NS
IB = 80
KB = 8
CH = KB * IB


def _mesh():
    return plsc.VectorSubcoreMesh(
        core_axis_name="c", subcore_axis_name="s",
        num_cores=NC, num_subcores=NS)


def _worker_units(total_units):
    w = lax.axis_index("s") * NC + lax.axis_index("c")
    q, r = divmod(total_units, NW)
    u0 = w * q + jnp.minimum(w, r)
    nu = q + jnp.where(w < r, 1, 0)
    return u0, nu


def _gather_motif(motif, src2, dst2):
    N, M = motif.shape
    E = src2.shape[0] * IB
    UNITS = E // CH

    @functools.partial(
        pl.kernel,
        out_type=(jax.ShapeDtypeStruct((E, M), jnp.float32),
                  jax.ShapeDtypeStruct((E, M), jnp.float32)),
        mesh=_mesh(),
        compiler_params=pltpu.CompilerParams(use_tc_tiling_on_sc=False),
        scratch_types=[
            pltpu.VMEM((KB, IB), jnp.int32),
            pltpu.VMEM((KB, IB), jnp.int32),
            pltpu.VMEM((CH, M), jnp.float32),
            pltpu.VMEM((CH, M), jnp.float32),
            pltpu.SemaphoreType.DMA,
            pltpu.SemaphoreType.DMA,
        ],
    )
    def k(motif_hbm, src_hbm, dst_hbm, mu_hbm, mv_hbm,
          sidx, didx, mrow, nrow, sem1, sem2):
        u0, nu = _worker_units(UNITS)

        @pl.loop(0, nu)
        def chunk(kk):
            orow = pl.multiple_of((u0 + kk) * KB, 8)
            off = orow * IB
            pltpu.sync_copy(src_hbm.at[pl.ds(orow, KB), :], sidx)
            pltpu.sync_copy(dst_hbm.at[pl.ds(orow, KB), :], didx)
            for j in range(KB):
                pltpu.async_copy(motif_hbm.at[sidx.at[j]],
                                 mrow.at[pl.ds(j * IB, IB), :], sem1)
                pltpu.async_copy(motif_hbm.at[didx.at[j]],
                                 nrow.at[pl.ds(j * IB, IB), :], sem2)
            for j in range(KB):
                pltpu.make_async_copy(motif_hbm.at[sidx.at[j]],
                                      mrow.at[pl.ds(j * IB, IB), :], sem1).wait()
                pltpu.make_async_copy(motif_hbm.at[didx.at[j]],
                                      nrow.at[pl.ds(j * IB, IB), :], sem2).wait()
            pltpu.sync_copy(mrow, mu_hbm.at[pl.ds(off, CH), :])
            pltpu.sync_copy(nrow, mv_hbm.at[pl.ds(off, CH), :])

    return k(motif, src2, dst2)


def _gate_tc(mu, mv, G1, g1b, G2, g2b):
    E, M = mu.shape
    BE = 3200
    grid = E // BE

    def body(mu_ref, mv_ref, G1_ref, g1b_ref, g2r_ref, g2b_ref, out_ref):
        a = mu_ref[...]
        b = mv_ref[...]
        feats = jnp.concatenate([a, b, jnp.abs(a - b), a * b], axis=1)
        h = jnp.dot(feats, G1_ref[...], preferred_element_type=jnp.float32)
        h = jnp.maximum(h + g1b_ref[...], 0.0)
        g = jnp.sum(h * g2r_ref[...], axis=1, keepdims=True) + g2b_ref[0, 0]
        g = jax.nn.sigmoid(g)
        out_ref[...] = jnp.clip(g, 0.0, 1.0)

    return pl.pallas_call(
        body,
        grid=(grid,),
        in_specs=[
            pl.BlockSpec((BE, M), lambda i: (i, 0)),
            pl.BlockSpec((BE, M), lambda i: (i, 0)),
            pl.BlockSpec((4 * M, 64), lambda i: (0, 0)),
            pl.BlockSpec((1, 64), lambda i: (0, 0)),
            pl.BlockSpec((1, 64), lambda i: (0, 0)),
            pl.BlockSpec((1, 1), lambda i: (0, 0)),
        ],
        out_specs=pl.BlockSpec((BE, 1), lambda i: (i, 0)),
        out_shape=jax.ShapeDtypeStruct((E, 1), jnp.float32),
    )(mu, mv, G1, g1b.reshape(1, 64), G2.reshape(1, 64), g2b.reshape(1, 1))


def _deg_sc(col2, gate2d, zeros1, NPAD):
    E = col2.shape[0] * IB
    UNITS = E // CH
    SL = NPAD // NS

    @functools.partial(
        pl.kernel,
        out_type=jax.ShapeDtypeStruct((NC, NPAD, 1), jnp.float32),
        mesh=_mesh(),
        compiler_params=pltpu.CompilerParams(use_tc_tiling_on_sc=False),
        scratch_types=[
            pltpu.VMEM((KB, IB), jnp.int32),
            pltpu.VMEM((CH, 1), jnp.float32),
            pltpu.VMEM_SHARED((NPAD, 1), jnp.float32),
        ],
    )
    def k(col_hbm, gate_hbm, z_hbm, out_hbm, cidx, gv, acc):
        cid = lax.axis_index("c")
        sid = lax.axis_index("s")
        r0 = sid * SL
        u0, nu = _worker_units(UNITS)
        pltpu.sync_copy(z_hbm.at[pl.ds(r0, SL), :], acc.at[pl.ds(r0, SL), :])
        plsc.subcore_barrier()

        @pl.loop(0, nu)
        def chunk(kk):
            orow = pl.multiple_of((u0 + kk) * KB, 8)
            off = orow * IB
            pltpu.sync_copy(col_hbm.at[pl.ds(orow, KB), :], cidx)
            pltpu.sync_copy(gate_hbm.at[pl.ds(off, CH), :], gv)
            for j in range(KB):
                pltpu.sync_copy(gv.at[pl.ds(j * IB, IB), :],
                                acc.at[cidx.at[j]], add=True)

        plsc.subcore_barrier()
        pltpu.sync_copy(acc.at[pl.ds(r0, SL), :],
                        out_hbm.at[cid, pl.ds(r0, SL), :])

    return k(col2, gate2d, zeros1)


def _dense1_tc(x, W1, res1_W, res1_b, degp, NPAD):
    N = x.shape[0]
    H = W1.shape[1]

    def body(x_ref, W1_ref, rW_ref, rb_ref, degp_ref, ys1_ref, xres_ref, dinv_ref):
        deg = degp_ref[0] + degp_ref[1] + 1.0
        dinv = lax.rsqrt(deg)
        dinv_ref[...] = dinv
        xw = jnp.dot(x_ref[...], W1_ref[...], preferred_element_type=jnp.float32)
        ys1_ref[...] = xw * dinv[:N]
        xres_ref[...] = (jnp.dot(x_ref[...], rW_ref[...],
                                 preferred_element_type=jnp.float32) + rb_ref[...])

    return pl.pallas_call(
        body,
        out_shape=(jax.ShapeDtypeStruct((N, H), jnp.float32),
                   jax.ShapeDtypeStruct((N, H), jnp.float32),
                   jax.ShapeDtypeStruct((NPAD, 1), jnp.float32)),
    )(x, W1, res1_W, res1_b.reshape(1, H), degp)


def _conv_sc(ys, row2, col2, gate_flat, zeros2, NPAD):
    N, H = ys.shape
    E = gate_flat.shape[0]
    UNITS = E // CH
    SL = NPAD // NS

    @functools.partial(
        pl.kernel,
        out_type=jax.ShapeDtypeStruct((NC, NPAD, H), jnp.float32),
        mesh=_mesh(),
        compiler_params=pltpu.CompilerParams(use_tc_tiling_on_sc=False),
        scratch_types=[
            pltpu.VMEM((KB, IB), jnp.int32),
            pltpu.VMEM((KB, IB), jnp.int32),
            pltpu.VMEM((CH,), jnp.float32),
            pltpu.VMEM((CH, H), jnp.float32),
            pltpu.VMEM_SHARED((NPAD, H), jnp.float32),
            pltpu.SemaphoreType.DMA,
        ],
    )
    def k(ys_hbm, row_hbm, col_hbm, gate_hbm, z_hbm, out_hbm,
          rid, cidx, gv, rows, acc, sem):
        cid_ax = lax.axis_index("c")
        sid = lax.axis_index("s")
        r0 = sid * SL
        u0, nu = _worker_units(UNITS)
        pltpu.sync_copy(z_hbm.at[pl.ds(r0, SL), :], acc.at[pl.ds(r0, SL), :])
        plsc.subcore_barrier()

        @pl.loop(0, nu)
        def chunk(kk):
            orow = pl.multiple_of((u0 + kk) * KB, 8)
            off = orow * IB
            pltpu.sync_copy(row_hbm.at[pl.ds(orow, KB), :], rid)
            pltpu.sync_copy(col_hbm.at[pl.ds(orow, KB), :], cidx)
            pltpu.sync_copy(gate_hbm.at[pl.ds(off, CH)], gv)
            for j in range(KB):
                pltpu.async_copy(ys_hbm.at[rid.at[j]],
                                 rows.at[pl.ds(j * IB, IB), :], sem)
            for j in range(KB):
                pltpu.make_async_copy(ys_hbm.at[rid.at[j]],
                                      rows.at[pl.ds(j * IB, IB), :], sem).wait()

            @pl.loop(0, CH // LANES)
            def scale(t):
                o = pl.multiple_of(t * LANES, 8)
                wvec = gv[pl.ds(o, LANES)]
                for i in range(LANES):
                    s = wvec[i]
                    for j in range(H // LANES):
                        rows[o + i, pl.ds(j * LANES, LANES)] = (
                            rows[o + i, pl.ds(j * LANES, LANES)] * s)

            for j in range(KB):
                pltpu.sync_copy(rows.at[pl.ds(j * IB, IB), :],
                                acc.at[cidx.at[j]], add=True)

        plsc.subcore_barrier()
        pltpu.sync_copy(acc.at[pl.ds(r0, SL), :],
                        out_hbm.at[cid_ax, pl.ds(r0, SL), :])

    return k(ys, row2, col2, gate_flat, zeros2)


def _post1_tc(accp, ys1, dinvc, b1, ln1_g, ln1_b, xres, W2):
    N, H = ys1.shape

    def body(accp_ref, ys1_ref, dinv_ref, b1_ref, g_ref, bb_ref, xres_ref,
             W2_ref, x1_ref, ys2_ref):
        dinv = dinv_ref[pl.ds(0, N), :]
        h = ((accp_ref[0, pl.ds(0, N), :] + accp_ref[1, pl.ds(0, N), :]
              + ys1_ref[...]) * dinv + b1_ref[...])
        mu = jnp.mean(h, axis=1, keepdims=True)
        var = jnp.mean((h - mu) ** 2, axis=1, keepdims=True)
        h = (h - mu) / jnp.sqrt(var + 1e-5) * g_ref[...] + bb_ref[...]
        h = jnp.maximum(h, 0.0)
        x1 = xres_ref[...] + h
        x1_ref[...] = x1
        ys2_ref[...] = jnp.dot(x1, W2_ref[...],
                               preferred_element_type=jnp.float32) * dinv

    return pl.pallas_call(
        body,
        out_shape=(jax.ShapeDtypeStruct((N, H), jnp.float32),
                   jax.ShapeDtypeStruct((N, H), jnp.float32)),
    )(accp, ys1, dinvc, b1.reshape(1, H), ln1_g.reshape(1, H),
      ln1_b.reshape(1, H), xres, W2)


def _final_tc(accp, ys2, dinvc, b2, ln2_g, ln2_b, x1, head_W, head_b):
    N, H = ys2.shape
    OUT = head_W.shape[1]

    def body(accp_ref, ys2_ref, dinv_ref, b2_ref, g_ref, bb_ref, x1_ref,
             hW_ref, hb_ref, out_ref):
        dinv = dinv_ref[pl.ds(0, N), :]
        h = ((accp_ref[0, pl.ds(0, N), :] + accp_ref[1, pl.ds(0, N), :]
              + ys2_ref[...]) * dinv + b2_ref[...])
        mu = jnp.mean(h, axis=1, keepdims=True)
        var = jnp.mean((h - mu) ** 2, axis=1, keepdims=True)
        h = (h - mu) / jnp.sqrt(var + 1e-5) * g_ref[...] + bb_ref[...]
        h = jnp.maximum(h, 0.0)
        x2 = x1_ref[...] + h
        out_ref[...] = (jnp.dot(x2, hW_ref[...],
                                preferred_element_type=jnp.float32) + hb_ref[...])

    return pl.pallas_call(
        body,
        out_shape=jax.ShapeDtypeStruct((N, OUT), jnp.float32),
    )(accp, ys2, dinvc, b2.reshape(1, H), ln2_g.reshape(1, H),
      ln2_b.reshape(1, H), x1, head_W, head_b.reshape(1, OUT))


def kernel(x, edge_index, motif_x, G1, g1b, G2, g2b, W1, b1, ln1_g, ln1_b,
           res1_W, res1_b, W2, b2, ln2_g, ln2_b, head_W, head_b):
    N = x.shape[0]
    E = edge_index.shape[1]
    H = W1.shape[1]
    NPAD = ((N + NS * 8 - 1) // (NS * 8)) * (NS * 8)

    row = edge_index[0]
    col = edge_index[1]
    row2 = row.reshape(E // IB, IB)
    col2 = col.reshape(E // IB, IB)
    zeros1 = jnp.zeros((NPAD, 1), jnp.float32)
    zeros2 = jnp.zeros((NPAD, H), jnp.float32)

    mu, mv = _gather_motif(motif_x, row2, col2)
    gate2d = _gate_tc(mu, mv, G1, g1b, G2, g2b)
    gate_flat = gate2d.reshape(E)
    degp = _deg_sc(col2, gate2d, zeros1, NPAD)
    ys1, xres, dinvc = _dense1_tc(x, W1, res1_W, res1_b, degp, NPAD)
    accp1 = _conv_sc(ys1, row2, col2, gate_flat, zeros2, NPAD)
    x1, ys2 = _post1_tc(accp1, ys1, dinvc, b1, ln1_g, ln1_b, xres, W2)
    accp2 = _conv_sc(ys2, row2, col2, gate_flat, zeros2, NPAD)
    return _final_tc(accp2, ys2, dinvc, b2, ln2_g, ln2_b, x1, head_W, head_b)
